# baseline (speedup 1.0000x reference)
import functools

import jax
import jax.numpy as jnp
from jax import lax
from jax.experimental import pallas as pl
from jax.experimental.pallas import tpu as pltpu
from jax.experimental.pallas import tpu_sc as plsc

_N, _E, _C, _D = 10000, 320000, 16, 128
_W = 160
_SCALED = _D + _C
_NC, _NS = 2, 16
_NW = _NC * _NS
_EPW = 10368
_EP = _EPW * _NW
_EP2 = _EP + 128
_K = 48
_NBLK = _EPW // _K
_NP = 10240
_RPT = _NP // _NS
_DUMMY = _N + 100


def _prologue_body(h_ref, lg_ref, al_ref, ar_ref, x_ref, el_ref, er_ref,
                   pred_ref):
    h = h_ref[...]
    el_ref[...] = jnp.sum(h * al_ref[...], axis=1, keepdims=True)
    er_ref[...] = jnp.sum(h * ar_ref[...], axis=1, keepdims=True)
    lg = lg_ref[...]
    pred = jnp.argmax(lg, axis=1).astype(jnp.int32)
    pred_ref[...] = pred[:, None]
    oh = (lax.broadcasted_iota(jnp.int32, (_N, _C), 1) == pred[:, None])
    x_ref[...] = jnp.concatenate(
        [h, oh.astype(jnp.float32),
         jnp.ones((_N, 1), jnp.float32),
         el_ref[...],
         jnp.zeros((_N, _W - _SCALED - 2), jnp.float32)], axis=1)


_prologue = pl.pallas_call(
    _prologue_body,
    out_shape=[
        jax.ShapeDtypeStruct((_N, _W), jnp.float32),
        jax.ShapeDtypeStruct((_N, 1), jnp.float32),
        jax.ShapeDtypeStruct((_N, 1), jnp.float32),
        jax.ShapeDtypeStruct((_N, 1), jnp.int32),
    ],
)


def _sc_edge_body(x_hbm, er_hbm, src_hbm, dst_hbm, out_hbm,
                  srcc, dstc, erb, rows, er_tab, acc,
                  sg0, sg1, ss0, ss1,
                  sis0, sis1, sis2, sis3, sid0, sid1, sid2, sid3):
    c = lax.axis_index("c")
    s = lax.axis_index("s")
    wid = s * _NC + c
    sg = (sg0, sg1)
    ss = (ss0, ss1)
    sis = (sis0, sis1, sis2, sis3)
    sid = (sid0, sid1, sid2, sid3)

    def _wait_rows(sem, p2):
        pltpu.make_async_copy(x_hbm.at[pl.ds(0, _K)], rows.at[p2], sem).wait()

    def _wait_idx(sem, ref, sl):
        pltpu.make_async_copy(src_hbm.at[pl.ds(0, _K)], ref.at[sl], sem).wait()

    pltpu.sync_copy(er_hbm, er_tab)

    def _erb_block(q2, sl1):
        for v in range(_K // 16):
            sl = pl.ds(v * 16, 16)
            erb[q2, sl] = plsc.load_gather(er_tab, [dstc[sl1, sl]])

    zv = jnp.zeros((16,), jnp.float32)

    def _zb(i, carry):
        for r in range(_W // 16):
            rows[1, i, pl.ds(r * 16, 16)] = zv
        return carry

    lax.fori_loop(0, _K, _zb, 0)
    for j in range(_RPT // _K):
        pltpu.sync_copy(rows.at[1], acc.at[pl.ds(s * _RPT + j * _K, _K)])
    pltpu.sync_copy(rows.at[1, pl.ds(0, _RPT % _K)],
                    acc.at[pl.ds(s * _RPT + (_RPT // _K) * _K, _RPT % _K)])
    plsc.subcore_barrier()

    base0 = wid * _EPW
    pltpu.async_copy(rows.at[1], acc.at[pl.ds(_N + 112, _K)], ss1)
    pltpu.async_copy(src_hbm.at[pl.ds(base0, _K)], srcc.at[0], sis0).wait()
    pltpu.async_copy(dst_hbm.at[pl.ds(base0, _K)], dstc.at[0], sid0).wait()
    pltpu.async_copy(src_hbm.at[pl.ds(base0 + _K, _K)], srcc.at[1], sis1)
    pltpu.async_copy(dst_hbm.at[pl.ds(base0 + _K, _K)], dstc.at[1], sid1)
    pltpu.async_copy(x_hbm.at[srcc.at[0]], rows.at[0], sg0)
    _erb_block(0, 0)

    def _scale_block(p2):
        def _scale(i, carry2):
            ev = rows[p2, i, pl.ds(_SCALED, 16)]
            rv = erb[p2, pl.ds(i, 16)]
            el_s = ev[1]
            er_d = rv[0]
            xx = el_s + er_d
            e = jnp.where(xx >= 0.0, xx, 0.2 * xx)
            sh = jnp.where(er_d >= 0.0, er_d, 0.2 * er_d)
            g = jnp.exp(jnp.full((16,), e - sh, jnp.float32))
            for r in range(_SCALED // 16):
                rows[p2, i, pl.ds(r * 16, 16)] = (
                    rows[p2, i, pl.ds(r * 16, 16)] * g)
            return carry2

        lax.fori_loop(0, _K, _scale, 0)

    def _super(i, carry):
        for p in range(4):
            p2 = p % 2
            q2 = 1 - p2
            b = i * 4 + p
            _wait_rows(sg[p2], p2)
            _scale_block(p2)
            pltpu.async_copy(rows.at[p2], acc.at[dstc.at[p]], ss[p2],
                             add=True)
            _wait_rows(ss[q2], q2)
            sl2 = (p + 2) % 4
            off2 = base0 + (b + 2) * _K
            pltpu.async_copy(src_hbm.at[pl.ds(off2, _K)], srcc.at[sl2],
                             sis[sl2])
            pltpu.async_copy(dst_hbm.at[pl.ds(off2, _K)], dstc.at[sl2],
                             sid[sl2])
            sl1 = (p + 1) % 4
            _wait_idx(sis[sl1], srcc, sl1)
            _wait_idx(sid[sl1], dstc, sl1)
            pltpu.async_copy(x_hbm.at[srcc.at[sl1]], rows.at[q2], sg[q2])
            _erb_block(q2, sl1)
        return carry

    lax.fori_loop(0, _NBLK // 4, _super, 0)
    _wait_rows(ss[1], 1)
    _wait_rows(sg[0], 0)
    _wait_idx(sis[1], srcc, 1)
    _wait_idx(sid[1], dstc, 1)
    plsc.subcore_barrier()

    for j in range(_RPT // _K):
        sl = pl.ds(s * _RPT + j * _K, _K)
        pltpu.sync_copy(acc.at[sl], out_hbm.at[c, sl])
    slr = pl.ds(s * _RPT + (_RPT // _K) * _K, _RPT % _K)
    pltpu.sync_copy(acc.at[slr], out_hbm.at[c, slr])


@functools.lru_cache(maxsize=1)
def _sc_edge():
  return pl.kernel(
    _sc_edge_body,
    out_type=jax.ShapeDtypeStruct((_NC, _NP, _W), jnp.float32),
    mesh=plsc.VectorSubcoreMesh(core_axis_name="c", subcore_axis_name="s",
                                num_cores=_NC, num_subcores=_NS),
    scratch_types=(
        [
            pltpu.VMEM((4, _K), jnp.int32),
            pltpu.VMEM((4, _K), jnp.int32),
            pltpu.VMEM((2, _K + 16), jnp.float32),
            pltpu.VMEM((2, _K, _W), jnp.float32),
            pltpu.VMEM((_NP,), jnp.float32),
            pltpu.VMEM_SHARED((_NP, _W), jnp.float32),
        ]
        + [pltpu.SemaphoreType.DMA] * 12
    ),
    compiler_params=pltpu.CompilerParams(needs_layout_passes=False,
                                         use_tc_tiling_on_sc=False),
  )


def _stats_body(extT_ref, predT_ref, ozT_ref, t1_ref, t2_ref,
                zT_ref, coefT_ref):
    extT = extT_ref[0] + extT_ref[1]
    cu = extT[:_C]
    degs = jnp.maximum(extT[_C:_C + 1], 1.0)
    esum = jnp.sum(cu, axis=0, keepdims=True)
    se = jnp.maximum(esum, 1e-16)
    cnts = cu / se / degs
    predT = predT_ref[...]
    oh = (lax.broadcasted_iota(jnp.int32, (_C, _N), 0) == predT)
    f1 = jnp.sum(jnp.where(oh, cnts, 0.0), axis=0, keepdims=True)
    present = jnp.sum(cu, axis=1, keepdims=True) > 0.0
    cc = jnp.maximum(cnts, 1e-5)
    f2 = -jnp.sum(jnp.where(present, cc * jnp.log(cc), 0.0), axis=0,
                  keepdims=True)

    def _ln(x):
        mu = jnp.mean(x)
        var = jnp.mean((x - mu) ** 2)
        return (x - mu) / jnp.sqrt(var + 1e-5)

    def _sig(x):
        return 1.0 / (1.0 + jnp.exp(-x))

    z = _sig(-(_ln(f1) - t1_ref[0, 0])) * _sig(-(_ln(f2) - t2_ref[0, 0]))
    zT_ref[...] = z
    coefT_ref[...] = jnp.minimum(ozT_ref[...], z) * lax.rsqrt(degs) / se


_stats = pl.pallas_call(
    _stats_body,
    out_shape=[
        jax.ShapeDtypeStruct((1, _N), jnp.float32),
        jax.ShapeDtypeStruct((1, _N), jnp.float32),
    ],
)


def _update_body(h_ref, agg_ref, zT_ref, coefT_ref, nh_ref, z_ref):
    coef = jnp.transpose(coefT_ref[...])
    nh_ref[...] = h_ref[...] + coef * (agg_ref[0] + agg_ref[1])
    z_ref[...] = jnp.transpose(zT_ref[...])


_update = pl.pallas_call(
    _update_body,
    out_shape=[
        jax.ShapeDtypeStruct((_N, _D), jnp.float32),
        jax.ShapeDtypeStruct((_N, 1), jnp.float32),
    ],
)


def kernel(h, logits, old_z, attn_l, attn_r, tau1, tau2, edge_index):
    nh_, hh, dd = h.shape
    h2 = h.reshape(nh_, dd)
    x, el, er, pred = _prologue(h2, logits,
                                attn_l.reshape(1, dd), attn_r.reshape(1, dd))
    pad = _EP2 - _E
    srcp = jnp.concatenate([edge_index[0],
                            jnp.zeros((pad,), edge_index.dtype)])
    dstp = jnp.concatenate([edge_index[1],
                            jnp.full((pad,), _DUMMY, edge_index.dtype)])
    erp = jnp.concatenate([er.reshape(_N),
                           jnp.zeros((_NP - _N,), jnp.float32)])
    ext = _sc_edge()(x, erp, srcp, dstp)
    extT = jnp.transpose(ext[:, :_N, _D:_SCALED + 1], (0, 2, 1))
    agg = ext[:, :_N, :_D]
    zT, coefT = _stats(extT, pred.reshape(1, _N), old_z.reshape(1, _N),
                       tau1.reshape(1, 1), tau2.reshape(1, 1))
    nh, z = _update(h2, agg, zT, coefT)
    return nh.reshape(nh_, hh, dd), z


# speedup vs baseline: 1.6190x; 1.6190x over previous
import functools

import jax
import jax.numpy as jnp
from jax import lax
from jax.experimental import pallas as pl
from jax.experimental.pallas import tpu as pltpu
from jax.experimental.pallas import tpu_sc as plsc

_N, _E, _C, _D = 10000, 320000, 16, 128
_W = 160
_SCALED = _D + _C
_NC, _NS = 2, 16
_NW = _NC * _NS
_EPW = _E // _NW
_K = 80
_NBLK = _EPW // _K
_NP = 10240
_RPT = _NP // _NS
_ZR = 128


def _prologue_body(h_ref, lg_ref, al_ref, ar_ref, x_ref, el_ref, er_ref,
                   pred_ref):
    h = h_ref[...]
    el_ref[...] = jnp.sum(h * al_ref[...], axis=1, keepdims=True)
    er_ref[...] = jnp.sum(h * ar_ref[...], axis=1, keepdims=True)
    lg = lg_ref[...]
    pred = jnp.argmax(lg, axis=1).astype(jnp.int32)
    pred_ref[...] = pred[:, None]
    oh = (lax.broadcasted_iota(jnp.int32, (_N, _C), 1) == pred[:, None])
    x_ref[...] = jnp.concatenate(
        [h, oh.astype(jnp.float32),
         jnp.ones((_N, 1), jnp.float32),
         el_ref[...],
         jnp.zeros((_N, _W - _SCALED - 2), jnp.float32)], axis=1)


_prologue = pl.pallas_call(
    _prologue_body,
    out_shape=[
        jax.ShapeDtypeStruct((_N, _W), jnp.float32),
        jax.ShapeDtypeStruct((_N, 1), jnp.float32),
        jax.ShapeDtypeStruct((_N, 1), jnp.float32),
        jax.ShapeDtypeStruct((_N, 1), jnp.int32),
    ],
)


def _sc_edge_body(x_hbm, er_hbm, src_hbm, dst_hbm, out_hbm,
                  er_tab, srcb, dstb, erb, rows, acc, sem):
    c = lax.axis_index("c")
    s = lax.axis_index("s")
    wid = s * _NC + c

    pltpu.sync_copy(er_hbm, er_tab)

    zv = jnp.zeros((16,), jnp.float32)

    def _zb(i, carry):
        for r in range(_W // 16):
            rows[i, pl.ds(r * 16, 16)] = zv
        return carry

    lax.fori_loop(0, _K, _zb, 0)
    for j in range(_RPT // _K):
        pltpu.sync_copy(rows, acc.at[pl.ds(s * _RPT + j * _K, _K)])
    plsc.subcore_barrier()

    base0 = wid * _EPW

    def _blk(b, carry):
        off = base0 + b * _K
        pltpu.sync_copy(src_hbm.at[pl.ds(off, _K)], srcb)
        pltpu.sync_copy(dst_hbm.at[pl.ds(off, _K)], dstb)
        cp = pltpu.async_copy(x_hbm.at[srcb], rows, sem)
        for v in range(_K // 16):
            sl = pl.ds(v * 16, 16)
            erb[sl] = plsc.load_gather(er_tab, [dstb[sl]])
        cp.wait()

        def _scale(i, carry2):
            ev = rows[i, pl.ds(_SCALED, 16)]
            rv = erb[pl.ds(i, 16)]
            el_s = ev[1]
            er_d = rv[0]
            xx = el_s + er_d
            e = jnp.where(xx >= 0.0, xx, 0.2 * xx)
            sh = jnp.where(er_d >= 0.0, er_d, 0.2 * er_d)
            g = jnp.exp(jnp.full((16,), e - sh, jnp.float32))
            for r in range(_SCALED // 16):
                rows[i, pl.ds(r * 16, 16)] = rows[i, pl.ds(r * 16, 16)] * g
            return carry2

        lax.fori_loop(0, _K, _scale, 0)
        pltpu.sync_copy(rows, acc.at[dstb], add=True)
        return carry

    lax.fori_loop(0, _NBLK, _blk, 0)
    plsc.subcore_barrier()

    for j in range(_RPT // _ZR):
        sl = pl.ds(s * _RPT + j * _ZR, _ZR)
        pltpu.sync_copy(acc.at[sl], out_hbm.at[c, sl])


@functools.lru_cache(maxsize=1)
def _sc_edge():
  return pl.kernel(
    _sc_edge_body,
    out_type=jax.ShapeDtypeStruct((_NC, _NP, _W), jnp.float32),
    mesh=plsc.VectorSubcoreMesh(core_axis_name="c", subcore_axis_name="s",
                                num_cores=_NC, num_subcores=_NS),
    scratch_types=[
        pltpu.VMEM((_N,), jnp.float32),
        pltpu.VMEM((_K,), jnp.int32),
        pltpu.VMEM((_K,), jnp.int32),
        pltpu.VMEM((_K + 16,), jnp.float32),
        pltpu.VMEM((_K, _W), jnp.float32),
        pltpu.VMEM_SHARED((_NP, _W), jnp.float32),
        pltpu.SemaphoreType.DMA,
    ],
    compiler_params=pltpu.CompilerParams(needs_layout_passes=False,
                                         use_tc_tiling_on_sc=False),
  )


def _stats_body(extT_ref, predT_ref, ozT_ref, t1_ref, t2_ref,
                zT_ref, coefT_ref):
    extT = extT_ref[0] + extT_ref[1]
    cu = extT[:_C]
    degs = jnp.maximum(extT[_C:_C + 1], 1.0)
    esum = jnp.sum(cu, axis=0, keepdims=True)
    se = jnp.maximum(esum, 1e-16)
    cnts = cu / se / degs
    predT = predT_ref[...]
    oh = (lax.broadcasted_iota(jnp.int32, (_C, _N), 0) == predT)
    f1 = jnp.sum(jnp.where(oh, cnts, 0.0), axis=0, keepdims=True)
    present = jnp.sum(cu, axis=1, keepdims=True) > 0.0
    cc = jnp.maximum(cnts, 1e-5)
    f2 = -jnp.sum(jnp.where(present, cc * jnp.log(cc), 0.0), axis=0,
                  keepdims=True)

    def _ln(x):
        mu = jnp.mean(x)
        var = jnp.mean((x - mu) ** 2)
        return (x - mu) / jnp.sqrt(var + 1e-5)

    def _sig(x):
        return 1.0 / (1.0 + jnp.exp(-x))

    z = _sig(-(_ln(f1) - t1_ref[0, 0])) * _sig(-(_ln(f2) - t2_ref[0, 0]))
    zT_ref[...] = z
    coefT_ref[...] = jnp.minimum(ozT_ref[...], z) * lax.rsqrt(degs) / se


_stats = pl.pallas_call(
    _stats_body,
    out_shape=[
        jax.ShapeDtypeStruct((1, _N), jnp.float32),
        jax.ShapeDtypeStruct((1, _N), jnp.float32),
    ],
)


def _update_body(h_ref, agg_ref, zT_ref, coefT_ref, nh_ref, z_ref):
    coef = jnp.transpose(coefT_ref[...])
    nh_ref[...] = h_ref[...] + coef * (agg_ref[0] + agg_ref[1])
    z_ref[...] = jnp.transpose(zT_ref[...])


_update = pl.pallas_call(
    _update_body,
    out_shape=[
        jax.ShapeDtypeStruct((_N, _D), jnp.float32),
        jax.ShapeDtypeStruct((_N, 1), jnp.float32),
    ],
)


def kernel(h, logits, old_z, attn_l, attn_r, tau1, tau2, edge_index):
    nh_, hh, dd = h.shape
    h2 = h.reshape(nh_, dd)
    x, el, er, pred = _prologue(h2, logits,
                                attn_l.reshape(1, dd), attn_r.reshape(1, dd))
    ext = _sc_edge()(x, er.reshape(_N), edge_index[0], edge_index[1])
    extT = jnp.transpose(ext[:, :_N, _D:_SCALED + 1], (0, 2, 1))
    agg = ext[:, :_N, :_D]
    zT, coefT = _stats(extT, pred.reshape(1, _N), old_z.reshape(1, _N),
                       tau1.reshape(1, 1), tau2.reshape(1, 1))
    nh, z = _update(h2, agg, zT, coefT)
    return nh.reshape(nh_, hh, dd), z
